# baseline (speedup 1.0000x reference)
import functools

import jax
import jax.numpy as jnp
from jax import lax
from jax.experimental import pallas as pl
from jax.experimental.pallas import tpu as pltpu
from jax.experimental.pallas import tpu_sc as plsc

NC = 2
NS = 16
L = 16
NW = NC * NS

TCOLS = 7813
SHARD = 245
WINT = 6
WCOLS = WINT * 128
NWIN = 42
MAXT = TCOLS - WINT
NSUP = 7
SUPT = 36
MEMCAP = 768
SUPCAP = 160
EXTCAP = 96
ROWSLOTS = 32
BATCH = 16384
NCHUNK = BATCH // L


def _extract_body(d_model,
                  u_ids_hbm, i_ids_hbm, uembT_hbm, iembT_hbm,
                  ugath_hbm, igath_hbm,
                  ids_v, mem_id, mem_pos, sup_id, sup_pos,
                  win0, win1, ext_col, ext_pos, rowstage,
                  scnt_smem, wsem0, wsem1, rsem):
  wid = lax.axis_index("s") * NC + lax.axis_index("c")
  sid = lax.axis_index("s")
  lanes = lax.iota(jnp.int32, L)
  wins = (win0.at[sid], win1.at[sid])
  wsems = (wsem0, wsem1)

  def run_table(table_hbm, ids_hbm, out_hbm, etot0):
    shard_t0 = wid * SHARD
    lo_s = shard_t0 * 128
    hi_s = jnp.minimum((shard_t0 + SHARD) * 128, 1000000)

    def tstart(w):
      return jnp.minimum(shard_t0 + w * WINT, MAXT)

    def fire(w, k):
      off = pl.multiple_of(tstart(w) * 128, 128)
      pltpu.async_copy(table_hbm.at[:, pl.ds(off, WCOLS)], wins[k], wsems[k])

    def drain_win(k):
      pltpu.make_async_copy(
          table_hbm.at[:, pl.ds(0, WCOLS)], wins[k], wsems[k]).wait()

    fire(0, 0)
    fire(1, 1)

    pltpu.sync_copy(ids_hbm, ids_v.at[pl.ds(0, BATCH)])

    def scan_chunk(ch, cnt):
      ids_c = ids_v[pl.ds(ch * L, L)]
      m = (ids_c >= lo_s) & (ids_c < hi_s)
      plsc.store_compressed(mem_id.at[pl.ds(cnt, L)], ids_c, mask=m)
      plsc.store_compressed(mem_pos.at[pl.ds(cnt, L)], ch * L + lanes, mask=m)
      return cnt + plsc.all_reduce_population_count(m)[0]

    cnt = jnp.int32(0)
    nmemchunk = (cnt + L - 1) // L

    for b in range(NSUP):
      blo = lo_s + b * SUPT * 128
      bhi = lo_s + (b + 1) * SUPT * 128

      def sup_chunk(j, sc, blo=blo, bhi=bhi, b=b):
        ids_c = mem_id[pl.ds(j * L, L)]
        pos_c = mem_pos[pl.ds(j * L, L)]
        m = (ids_c >= blo) & (ids_c < bhi) & (j * L + lanes < cnt)
        plsc.store_compressed(sup_id.at[pl.ds(b * SUPCAP + sc, L)], ids_c, mask=m)
        plsc.store_compressed(sup_pos.at[pl.ds(b * SUPCAP + sc, L)], pos_c, mask=m)
        return sc + plsc.all_reduce_population_count(m)[0]

      scnt_smem[b] = lax.fori_loop(0, nmemchunk, sup_chunk, jnp.int32(0))

    def process(w, k, etot_in):
      lo = tstart(w) * 128
      sup = (w * WINT) // SUPT

      n_s = scnt_smem[sup]

      def rescan(j, ec):
        ids_c = sup_id[pl.ds(sup * SUPCAP + j * L, L)]
        pos_c = sup_pos[pl.ds(sup * SUPCAP + j * L, L)]
        m = (ids_c >= lo) & (ids_c < lo + WCOLS) & (j * L + lanes < n_s)
        plsc.store_compressed(ext_col.at[pl.ds(ec, L)], ids_c - lo, mask=m)
        plsc.store_compressed(ext_pos.at[pl.ds(ec, L)], pos_c, mask=m)
        return ec + plsc.all_reduce_population_count(m)[0]

      ecnt = jnp.int32(0)

      def extract(e, etot):
        c0 = ext_col[pl.ds(e, L)][0]
        b0 = ext_pos[pl.ds(e, L)][0]
        slot = (etot % ROWSLOTS) * d_model

        @pl.when(etot >= ROWSLOTS)
        def _():
          pltpu.make_async_copy(
              rowstage.at[pl.ds(0, d_model)],
              out_hbm.at[pl.ds(0, d_model)], rsem).wait()

        for dblk in range(d_model // L):
          g = plsc.load_gather(
              wins[k], [dblk * L + lanes, lanes * 0 + c0])
          rowstage[pl.ds(slot + dblk * L, L)] = g
        pltpu.async_copy(
            rowstage.at[pl.ds(slot, d_model)],
            out_hbm.at[pl.ds(b0 * d_model, d_model)], rsem)
        return etot + 1

      return etot_in + ecnt * 0

    def pair(p, etot):
      for k in range(2):
        w = p * 2 + k
        drain_win(k)
        etot = process(w, k, etot)
        fire(w + 2, k)
      return etot

    etot = lax.fori_loop(0, NWIN // 2 - 1, pair, etot0)
    for k in range(2):
      w = NWIN - 2 + k
      drain_win(k)
      etot = process(w, k, etot)
    return etot

  etot = run_table(uembT_hbm, u_ids_hbm, ugath_hbm, jnp.int32(0))
  etot = run_table(iembT_hbm, i_ids_hbm, igath_hbm, etot)

  def drain_row(j, carry):
    pltpu.make_async_copy(
        rowstage.at[pl.ds(0, d_model)],
        ugath_hbm.at[pl.ds(0, d_model)], rsem).wait()
    return carry

  lax.fori_loop(0, jnp.minimum(etot, ROWSLOTS), drain_row, jnp.int32(0))


def _dot_body(b_per_w, d_model,
              u_ids_hbm, i_ids_hbm, ugath_hbm, igath_hbm,
              user_bias_hbm, item_bias_hbm, bias_hbm, out_hbm,
              uid_v, iid_v, ug_v, ig_v, ub_v, ib_v, bias_v, out_v,
              sem_rows, sem_bias):
  wid = lax.axis_index("s") * NC + lax.axis_index("c")
  base = wid * b_per_w

  pltpu.sync_copy(u_ids_hbm.at[pl.ds(base, b_per_w)], uid_v)
  pltpu.sync_copy(i_ids_hbm.at[pl.ds(base, b_per_w)], iid_v)

  cp_u = pltpu.async_copy(
      ugath_hbm.at[pl.ds(base * d_model, b_per_w * d_model)], ug_v, sem_rows)
  cp_i = pltpu.async_copy(
      igath_hbm.at[pl.ds(base * d_model, b_per_w * d_model)], ig_v, sem_rows)
  cp_ub = pltpu.async_copy(user_bias_hbm.at[uid_v], ub_v, sem_bias)
  cp_ib = pltpu.async_copy(item_bias_hbm.at[iid_v], ib_v, sem_bias)
  pltpu.sync_copy(bias_hbm, bias_v)
  cp_u.wait()
  cp_i.wait()
  cp_ub.wait()
  cp_ib.wait()

  lanes = lax.iota(jnp.int32, L)
  bias_splat = bias_v[...]

  def group(g, carry):
    row = g * L
    acc = ub_v[pl.ds(row, L)] + ib_v[pl.ds(row, L)] + bias_splat
    idx0 = (lanes + row) * d_model
    for d in range(d_model):
      acc = acc + (plsc.load_gather(ug_v, [idx0 + d]) *
                   plsc.load_gather(ig_v, [idx0 + d]))
    out_v[pl.ds(row, L)] = acc
    return carry

  lax.fori_loop(0, b_per_w // L, group, 0)

  pltpu.sync_copy(out_v, out_hbm.at[pl.ds(base, b_per_w)])


def kernel(u_ids, i_ids, user_emb, item_emb, user_bias, item_bias, bias):
  batch = u_ids.shape[0]
  d_model = user_emb.shape[1]
  b_per_w = batch // NW
  bias16 = jnp.broadcast_to(bias, (L,))
  uembT = user_emb.T
  iembT = item_emb.T

  mesh = plsc.VectorSubcoreMesh(core_axis_name="c", subcore_axis_name="s",
                                num_cores=NC, num_subcores=NS)

  extract = pl.kernel(
      functools.partial(_extract_body, d_model),
      out_type=(jax.ShapeDtypeStruct((batch * d_model,), jnp.float32),
                jax.ShapeDtypeStruct((batch * d_model,), jnp.float32)),
      mesh=mesh,
      compiler_params=pltpu.CompilerParams(needs_layout_passes=False),
      scratch_types=[
          pltpu.VMEM((BATCH + L,), jnp.int32),
          pltpu.VMEM((MEMCAP + L,), jnp.int32),
          pltpu.VMEM((MEMCAP + L,), jnp.int32),
          pltpu.VMEM((NSUP * SUPCAP + L,), jnp.int32),
          pltpu.VMEM((NSUP * SUPCAP + L,), jnp.int32),
          pltpu.VMEM_SHARED((NS, 64, WCOLS), jnp.float32),
          pltpu.VMEM_SHARED((NS, 64, WCOLS), jnp.float32),
          pltpu.VMEM((EXTCAP + L,), jnp.int32),
          pltpu.VMEM((EXTCAP + L,), jnp.int32),
          pltpu.VMEM((ROWSLOTS * 64,), jnp.float32),
          pltpu.SMEM((NSUP,), jnp.int32),
          pltpu.SemaphoreType.DMA,
          pltpu.SemaphoreType.DMA,
          pltpu.SemaphoreType.DMA,
      ],
  )
  ugath, igath = extract(u_ids, i_ids, uembT, iembT)

  dot = pl.kernel(
      functools.partial(_dot_body, b_per_w, d_model),
      out_type=jax.ShapeDtypeStruct((batch,), jnp.float32),
      mesh=mesh,
      compiler_params=pltpu.CompilerParams(needs_layout_passes=False),
      scratch_types=[
          pltpu.VMEM((b_per_w,), jnp.int32),
          pltpu.VMEM((b_per_w,), jnp.int32),
          pltpu.VMEM((b_per_w * d_model,), jnp.float32),
          pltpu.VMEM((b_per_w * d_model,), jnp.float32),
          pltpu.VMEM((b_per_w,), jnp.float32),
          pltpu.VMEM((b_per_w,), jnp.float32),
          pltpu.VMEM((L,), jnp.float32),
          pltpu.VMEM((b_per_w,), jnp.float32),
          pltpu.SemaphoreType.DMA,
          pltpu.SemaphoreType.DMA,
      ],
  )
  return dot(u_ids, i_ids, ugath, igath, user_bias, item_bias, bias16)


# speedup vs baseline: 1.1400x; 1.1400x over previous
import functools

import jax
import jax.numpy as jnp
from jax import lax
from jax.experimental import pallas as pl
from jax.experimental.pallas import tpu as pltpu
from jax.experimental.pallas import tpu_sc as plsc

NC = 2
NS = 16
L = 16
NW = NC * NS

TCOLS = 7813
SHARD = 245
WINT = 5
WCOLS = WINT * 128
NWIN = 50
MAXT = TCOLS - WINT
NSUP = 7
SUPT = 35
MEMCAP = 768
SUPCAP = 160
EXTCAP = 96
ROWSLOTS = 32
BATCH = 16384
NCHUNK = BATCH // L


def _extract_body(d_model,
                  u_ids_hbm, i_ids_hbm, uembT_hbm, iembT_hbm,
                  ugath_hbm, igath_hbm,
                  idsu_v, idsi_v, mu_id, mu_pos, mi_id, mi_pos,
                  supu_id, supu_pos, supi_id, supi_pos,
                  win0, win1, ext_col, ext_pos, rowstage,
                  scnt_smem, wsem0, wsem1, rsem):
  wid = lax.axis_index("s") * NC + lax.axis_index("c")
  lanes = lax.iota(jnp.int32, L)
  wins = (win0, win1)
  wsems = (wsem0, wsem1)

  shard_t0 = wid * SHARD
  lo_s = shard_t0 * 128
  hi_s = jnp.minimum((shard_t0 + SHARD) * 128, 1000000)

  def tstart(w):
    return jnp.minimum(shard_t0 + w * WINT, MAXT)

  def fire(table_hbm, w, k):
    off = pl.multiple_of(tstart(w) * 128, 128)
    pltpu.async_copy(table_hbm.at[:, pl.ds(off, WCOLS)], wins[k], wsems[k])

  def drain_win(k):
    pltpu.make_async_copy(
        uembT_hbm.at[:, pl.ds(0, WCOLS)], wins[k], wsems[k]).wait()

  fire(uembT_hbm, 0, 0)
  fire(uembT_hbm, 1, 1)

  pltpu.sync_copy(u_ids_hbm, idsu_v.at[pl.ds(0, BATCH)])
  pltpu.sync_copy(i_ids_hbm, idsi_v.at[pl.ds(0, BATCH)])

  def scan_chunk(ch, cnts):
    cu, ci = cnts
    pos = ch * L + lanes
    u_c = idsu_v[pl.ds(ch * L, L)]
    mu = (u_c >= lo_s) & (u_c < hi_s)
    plsc.store_compressed(mu_id.at[pl.ds(cu, L)], u_c, mask=mu)
    plsc.store_compressed(mu_pos.at[pl.ds(cu, L)], pos, mask=mu)
    i_c = idsi_v[pl.ds(ch * L, L)]
    mi = (i_c >= lo_s) & (i_c < hi_s)
    plsc.store_compressed(mi_id.at[pl.ds(ci, L)], i_c, mask=mi)
    plsc.store_compressed(mi_pos.at[pl.ds(ci, L)], pos, mask=mi)
    return (cu + plsc.all_reduce_population_count(mu)[0],
            ci + plsc.all_reduce_population_count(mi)[0])

  cntu, cnti = lax.fori_loop(0, NCHUNK, scan_chunk,
                             (jnp.int32(0), jnp.int32(0)), unroll=4)

  for t, (m_id, m_pos, s_id, s_pos, cnt) in enumerate(
      ((mu_id, mu_pos, supu_id, supu_pos, cntu),
       (mi_id, mi_pos, supi_id, supi_pos, cnti))):
    nmemchunk = (cnt + L - 1) // L
    for b in range(NSUP):
      blo = lo_s + b * SUPT * 128
      bhi = lo_s + (b + 1) * SUPT * 128

      def sup_chunk(j, sc, blo=blo, bhi=bhi, b=b,
                    m_id=m_id, m_pos=m_pos, s_id=s_id, s_pos=s_pos, cnt=cnt):
        ids_c = m_id[pl.ds(j * L, L)]
        pos_c = m_pos[pl.ds(j * L, L)]
        m = (ids_c >= blo) & (ids_c < bhi) & (j * L + lanes < cnt)
        plsc.store_compressed(s_id.at[pl.ds(b * SUPCAP + sc, L)], ids_c,
                              mask=m)
        plsc.store_compressed(s_pos.at[pl.ds(b * SUPCAP + sc, L)], pos_c,
                              mask=m)
        return sc + plsc.all_reduce_population_count(m)[0]

      scnt_smem[t * NSUP + b] = lax.fori_loop(0, nmemchunk, sup_chunk,
                                              jnp.int32(0))

  def run_table(table_hbm, out_hbm, s_id, s_pos, sbase, etot0):
    def process(w, k, etot_in):
      lo = tstart(w) * 128
      sup = (w * WINT) // SUPT

      n_s = scnt_smem[sbase + sup]

      def rescan(j, ec):
        ids_c = s_id[pl.ds(sup * SUPCAP + j * L, L)]
        pos_c = s_pos[pl.ds(sup * SUPCAP + j * L, L)]
        m = (ids_c >= lo) & (ids_c < lo + WCOLS) & (j * L + lanes < n_s)
        plsc.store_compressed(ext_col.at[pl.ds(ec, L)], ids_c - lo, mask=m)
        plsc.store_compressed(ext_pos.at[pl.ds(ec, L)], pos_c, mask=m)
        return ec + plsc.all_reduce_population_count(m)[0]

      ecnt = lax.fori_loop(0, (n_s + L - 1) // L, rescan, jnp.int32(0))

      def extract(e, etot):
        c0 = ext_col[pl.ds(e, L)][0]
        b0 = ext_pos[pl.ds(e, L)][0]
        slot = (etot % ROWSLOTS) * d_model

        @pl.when(etot >= ROWSLOTS)
        def _():
          pltpu.make_async_copy(
              rowstage.at[pl.ds(0, d_model)],
              out_hbm.at[pl.ds(0, d_model)], rsem).wait()

        for dblk in range(d_model // L):
          g = plsc.load_gather(
              wins[k], [dblk * L + lanes, lanes * 0 + c0])
          rowstage[pl.ds(slot + dblk * L, L)] = g
        pltpu.async_copy(
            rowstage.at[pl.ds(slot, d_model)],
            out_hbm.at[pl.ds(b0 * d_model, d_model)], rsem)
        return etot + 1

      return lax.fori_loop(0, ecnt, extract, etot_in)

    def pair(p, etot):
      for k in range(2):
        w = p * 2 + k
        drain_win(k)
        etot = process(w, k, etot)
        fire(table_hbm, w + 2, k)
      return etot

    etot = lax.fori_loop(0, NWIN // 2 - 1, pair, etot0)
    for k in range(2):
      w = NWIN - 2 + k
      drain_win(k)
      etot = process(w, k, etot)
    return etot

  etot = run_table(uembT_hbm, ugath_hbm, supu_id, supu_pos, 0, jnp.int32(0))
  fire(iembT_hbm, 0, 0)
  fire(iembT_hbm, 1, 1)
  etot = run_table(iembT_hbm, igath_hbm, supi_id, supi_pos, NSUP, etot)

  def drain_row(j, carry):
    pltpu.make_async_copy(
        rowstage.at[pl.ds(0, d_model)],
        ugath_hbm.at[pl.ds(0, d_model)], rsem).wait()
    return carry

  lax.fori_loop(0, jnp.minimum(etot, ROWSLOTS), drain_row, jnp.int32(0))


def _dot_body(b_per_w, d_model,
              u_ids_hbm, i_ids_hbm, ugath_hbm, igath_hbm,
              user_bias_hbm, item_bias_hbm, bias_hbm, out_hbm,
              uid_v, iid_v, ug_v, ig_v, ub_v, ib_v, bias_v, out_v,
              sem_rows, sem_bias):
  wid = lax.axis_index("s") * NC + lax.axis_index("c")
  base = wid * b_per_w

  pltpu.sync_copy(u_ids_hbm.at[pl.ds(base, b_per_w)], uid_v)
  pltpu.sync_copy(i_ids_hbm.at[pl.ds(base, b_per_w)], iid_v)

  cp_u = pltpu.async_copy(
      ugath_hbm.at[pl.ds(base * d_model, b_per_w * d_model)], ug_v, sem_rows)
  cp_i = pltpu.async_copy(
      igath_hbm.at[pl.ds(base * d_model, b_per_w * d_model)], ig_v, sem_rows)
  cp_ub = pltpu.async_copy(user_bias_hbm.at[uid_v], ub_v, sem_bias)
  cp_ib = pltpu.async_copy(item_bias_hbm.at[iid_v], ib_v, sem_bias)
  pltpu.sync_copy(bias_hbm, bias_v)
  cp_u.wait()
  cp_i.wait()
  cp_ub.wait()
  cp_ib.wait()

  lanes = lax.iota(jnp.int32, L)
  bias_splat = bias_v[...]

  def group(g, carry):
    row = g * L
    acc = ub_v[pl.ds(row, L)] + ib_v[pl.ds(row, L)] + bias_splat
    idx0 = (lanes + row) * d_model
    for d in range(d_model):
      acc = acc + (plsc.load_gather(ug_v, [idx0 + d]) *
                   plsc.load_gather(ig_v, [idx0 + d]))
    out_v[pl.ds(row, L)] = acc
    return carry

  lax.fori_loop(0, b_per_w // L, group, 0)

  pltpu.sync_copy(out_v, out_hbm.at[pl.ds(base, b_per_w)])


def kernel(u_ids, i_ids, user_emb, item_emb, user_bias, item_bias, bias):
  batch = u_ids.shape[0]
  d_model = user_emb.shape[1]
  b_per_w = batch // NW
  bias16 = jnp.broadcast_to(bias, (L,))
  uembT = user_emb.T
  iembT = item_emb.T

  mesh = plsc.VectorSubcoreMesh(core_axis_name="c", subcore_axis_name="s",
                                num_cores=NC, num_subcores=NS)

  extract = pl.kernel(
      functools.partial(_extract_body, d_model),
      out_type=(jax.ShapeDtypeStruct((batch * d_model,), jnp.float32),
                jax.ShapeDtypeStruct((batch * d_model,), jnp.float32)),
      mesh=mesh,
      compiler_params=pltpu.CompilerParams(needs_layout_passes=False),
      scratch_types=[
          pltpu.VMEM((BATCH,), jnp.int32),
          pltpu.VMEM((BATCH,), jnp.int32),
          pltpu.VMEM((MEMCAP + L,), jnp.int32),
          pltpu.VMEM((MEMCAP + L,), jnp.int32),
          pltpu.VMEM((MEMCAP + L,), jnp.int32),
          pltpu.VMEM((MEMCAP + L,), jnp.int32),
          pltpu.VMEM((NSUP * SUPCAP + L,), jnp.int32),
          pltpu.VMEM((NSUP * SUPCAP + L,), jnp.int32),
          pltpu.VMEM((NSUP * SUPCAP + L,), jnp.int32),
          pltpu.VMEM((NSUP * SUPCAP + L,), jnp.int32),
          pltpu.VMEM((64, WCOLS), jnp.float32),
          pltpu.VMEM((64, WCOLS), jnp.float32),
          pltpu.VMEM((EXTCAP + L,), jnp.int32),
          pltpu.VMEM((EXTCAP + L,), jnp.int32),
          pltpu.VMEM((ROWSLOTS * 64,), jnp.float32),
          pltpu.SMEM((2 * NSUP,), jnp.int32),
          pltpu.SemaphoreType.DMA,
          pltpu.SemaphoreType.DMA,
          pltpu.SemaphoreType.DMA,
      ],
  )
  ugath, igath = extract(u_ids, i_ids, uembT, iembT)

  dot = pl.kernel(
      functools.partial(_dot_body, b_per_w, d_model),
      out_type=jax.ShapeDtypeStruct((batch,), jnp.float32),
      mesh=mesh,
      compiler_params=pltpu.CompilerParams(needs_layout_passes=False),
      scratch_types=[
          pltpu.VMEM((b_per_w,), jnp.int32),
          pltpu.VMEM((b_per_w,), jnp.int32),
          pltpu.VMEM((b_per_w * d_model,), jnp.float32),
          pltpu.VMEM((b_per_w * d_model,), jnp.float32),
          pltpu.VMEM((b_per_w,), jnp.float32),
          pltpu.VMEM((b_per_w,), jnp.float32),
          pltpu.VMEM((L,), jnp.float32),
          pltpu.VMEM((b_per_w,), jnp.float32),
          pltpu.SemaphoreType.DMA,
          pltpu.SemaphoreType.DMA,
      ],
  )
  return dot(u_ids, i_ids, ugath, igath, user_bias, item_bias, bias16)


# speedup vs baseline: 1.2648x; 1.1094x over previous
import functools

import jax
import jax.numpy as jnp
from jax import lax
from jax.experimental import pallas as pl
from jax.experimental.pallas import tpu as pltpu
from jax.experimental.pallas import tpu_sc as plsc

NC = 2
NS = 16
L = 16
NW = NC * NS

TCOLS = 7813
SHARD = 245
WINT = 5
WCOLS = WINT * 128
NWIN = 50
MAXT = TCOLS - WINT
NSUP = 7
SUPT = 35
MEMCAP = 768
SUPCAP = 160
EXTCAP = 96
ROWSLOTS = 32
BATCH = 16384
HALF = BATCH // 2
NCHUNK = BATCH // L


def _extract_body(d_model, b_per_w,
                  u_ids_hbm, i_ids_hbm, uembT_hbm, iembT_hbm, bias_hbm,
                  user_bias_hbm, item_bias_hbm,
                  ugath_hbm, igath_hbm, bsum_hbm,
                  idsu_v, idsi_v, mu_id, mu_pos, mi_id, mi_pos,
                  supu_id, supu_pos, supi_id, supi_pos,
                  win0, win1, ext_col, ext_pos, rowstage,
                  ub_v, ib_v, bias_v,
                  scnt_smem, wsem0, wsem1, rsem, sem_bias):
  wid = lax.axis_index("s") * NC + lax.axis_index("c")
  lanes = lax.iota(jnp.int32, L)
  wins = (win0, win1)
  wsems = (wsem0, wsem1)

  shard_t0 = wid * SHARD
  lo_s = shard_t0 * 128
  hi_s = jnp.minimum((shard_t0 + SHARD) * 128, 1000000)

  def tstart(w):
    return jnp.minimum(shard_t0 + w * WINT, MAXT)

  def fire(table_hbm, w, k):
    off = pl.multiple_of(tstart(w) * 128, 128)
    pltpu.async_copy(table_hbm.at[:, pl.ds(off, WCOLS)], wins[k], wsems[k])

  def drain_win(k):
    pltpu.make_async_copy(
        uembT_hbm.at[:, pl.ds(0, WCOLS)], wins[k], wsems[k]).wait()

  fire(uembT_hbm, 0, 0)
  fire(uembT_hbm, 1, 1)

  pltpu.sync_copy(u_ids_hbm, idsu_v.at[pl.ds(0, BATCH)])
  pltpu.sync_copy(i_ids_hbm, idsi_v.at[pl.ds(0, BATCH)])

  bbase = wid * b_per_w
  cp_ub = pltpu.async_copy(
      user_bias_hbm.at[idsu_v.at[pl.ds(bbase, b_per_w)]], ub_v, sem_bias)
  cp_ib = pltpu.async_copy(
      item_bias_hbm.at[idsi_v.at[pl.ds(bbase, b_per_w)]], ib_v, sem_bias)
  pltpu.sync_copy(bias_hbm, bias_v)

  def scan_chunk(ch, cnts):
    cu, ci = cnts
    pos = ch * L + lanes
    u_c = idsu_v[pl.ds(ch * L, L)]
    mu = (u_c >= lo_s) & (u_c < hi_s)
    plsc.store_compressed(mu_id.at[pl.ds(cu, L)], u_c, mask=mu)
    plsc.store_compressed(mu_pos.at[pl.ds(cu, L)], pos, mask=mu)
    i_c = idsi_v[pl.ds(ch * L, L)]
    mi = (i_c >= lo_s) & (i_c < hi_s)
    plsc.store_compressed(mi_id.at[pl.ds(ci, L)], i_c, mask=mi)
    plsc.store_compressed(mi_pos.at[pl.ds(ci, L)], pos, mask=mi)
    return (cu + plsc.all_reduce_population_count(mu)[0],
            ci + plsc.all_reduce_population_count(mi)[0])

  cntu, cnti = lax.fori_loop(0, NCHUNK, scan_chunk,
                             (jnp.int32(0), jnp.int32(0)), unroll=4)

  for t, (m_id, m_pos, s_id, s_pos, cnt) in enumerate(
      ((mu_id, mu_pos, supu_id, supu_pos, cntu),
       (mi_id, mi_pos, supi_id, supi_pos, cnti))):
    nmemchunk = (cnt + L - 1) // L
    for b in range(NSUP):
      blo = lo_s + b * SUPT * 128
      bhi = lo_s + (b + 1) * SUPT * 128

      def sup_chunk(j, sc, blo=blo, bhi=bhi, b=b,
                    m_id=m_id, m_pos=m_pos, s_id=s_id, s_pos=s_pos, cnt=cnt):
        ids_c = m_id[pl.ds(j * L, L)]
        pos_c = m_pos[pl.ds(j * L, L)]
        m = (ids_c >= blo) & (ids_c < bhi) & (j * L + lanes < cnt)
        plsc.store_compressed(s_id.at[pl.ds(b * SUPCAP + sc, L)], ids_c,
                              mask=m)
        plsc.store_compressed(s_pos.at[pl.ds(b * SUPCAP + sc, L)], pos_c,
                              mask=m)
        return sc + plsc.all_reduce_population_count(m)[0]

      scnt_smem[t * NSUP + b] = lax.fori_loop(0, nmemchunk, sup_chunk,
                                              jnp.int32(0))

  def run_table(table_hbm, out_hbm, s_id, s_pos, sbase, etot0):
    def process(w, k, etot_in):
      lo = tstart(w) * 128
      sup = (w * WINT) // SUPT

      n_s = scnt_smem[sbase + sup]

      def rescan(j, ec):
        ids_c = s_id[pl.ds(sup * SUPCAP + j * L, L)]
        pos_c = s_pos[pl.ds(sup * SUPCAP + j * L, L)]
        m = (ids_c >= lo) & (ids_c < lo + WCOLS) & (j * L + lanes < n_s)
        plsc.store_compressed(ext_col.at[pl.ds(ec, L)], ids_c - lo, mask=m)
        plsc.store_compressed(ext_pos.at[pl.ds(ec, L)], pos_c, mask=m)
        return ec + plsc.all_reduce_population_count(m)[0]

      ecnt = lax.fori_loop(0, (n_s + L - 1) // L, rescan, jnp.int32(0))

      def extract(e, etot):
        c0 = ext_col[pl.ds(e, L)][0]
        b0 = ext_pos[pl.ds(e, L)][0]
        slot = (etot % ROWSLOTS) * d_model

        @pl.when(etot >= ROWSLOTS)
        def _():
          pltpu.make_async_copy(
              rowstage.at[pl.ds(0, d_model)],
              out_hbm.at[pl.ds(0, d_model)], rsem).wait()

        for dblk in range(d_model // L):
          g = plsc.load_gather(
              wins[k], [dblk * L + lanes, lanes * 0 + c0])
          rowstage[pl.ds(slot + dblk * L, L)] = g
        foff = (b0 % HALF) * (2 * d_model) + (b0 // HALF) * d_model
        pltpu.async_copy(
            rowstage.at[pl.ds(slot, d_model)],
            out_hbm.at[pl.ds(foff, d_model)], rsem)
        return etot + 1

      return lax.fori_loop(0, ecnt, extract, etot_in)

    def pair(p, etot):
      for k in range(2):
        w = p * 2 + k
        drain_win(k)
        etot = process(w, k, etot)
        fire(table_hbm, w + 2, k)
      return etot

    etot = lax.fori_loop(0, NWIN // 2 - 1, pair, etot0)
    for k in range(2):
      w = NWIN - 2 + k
      drain_win(k)
      etot = process(w, k, etot)
    return etot

  cp_ub.wait()
  cp_ib.wait()
  bias_splat = bias_v[...]
  for c in range(b_per_w // L):
    sl = pl.ds(c * L, L)
    ub_v[sl] = ub_v[sl] + ib_v[sl] + bias_splat
  pltpu.sync_copy(ub_v, bsum_hbm.at[pl.ds(bbase, b_per_w)])

  etot = run_table(uembT_hbm, ugath_hbm, supu_id, supu_pos, 0, jnp.int32(0))
  fire(iembT_hbm, 0, 0)
  fire(iembT_hbm, 1, 1)
  etot = run_table(iembT_hbm, igath_hbm, supi_id, supi_pos, NSUP, etot)

  def drain_row(j, carry):
    pltpu.make_async_copy(
        rowstage.at[pl.ds(0, d_model)],
        ugath_hbm.at[pl.ds(0, d_model)], rsem).wait()
    return carry

  lax.fori_loop(0, jnp.minimum(etot, ROWSLOTS), drain_row, jnp.int32(0))


def _tc_dot_body(d_model, u_ref, i_ref, b_ref, o_ref):
  prod = u_ref[...] * i_ref[...]
  a = jnp.sum(prod[:, :d_model], axis=1)
  b = jnp.sum(prod[:, d_model:], axis=1)
  o_ref[...] = b_ref[...] + jnp.stack([a, b], axis=0)


def kernel(u_ids, i_ids, user_emb, item_emb, user_bias, item_bias, bias):
  batch = u_ids.shape[0]
  d_model = user_emb.shape[1]
  b_per_w = batch // NW
  bias16 = jnp.broadcast_to(bias, (L,))
  uembT = user_emb.T
  iembT = item_emb.T

  mesh = plsc.VectorSubcoreMesh(core_axis_name="c", subcore_axis_name="s",
                                num_cores=NC, num_subcores=NS)

  extract = pl.kernel(
      functools.partial(_extract_body, d_model, b_per_w),
      out_type=(jax.ShapeDtypeStruct((batch * d_model,), jnp.float32),
                jax.ShapeDtypeStruct((batch * d_model,), jnp.float32),
                jax.ShapeDtypeStruct((batch,), jnp.float32)),
      mesh=mesh,
      compiler_params=pltpu.CompilerParams(needs_layout_passes=False),
      scratch_types=[
          pltpu.VMEM((BATCH,), jnp.int32),
          pltpu.VMEM((BATCH,), jnp.int32),
          pltpu.VMEM((MEMCAP + L,), jnp.int32),
          pltpu.VMEM((MEMCAP + L,), jnp.int32),
          pltpu.VMEM((MEMCAP + L,), jnp.int32),
          pltpu.VMEM((MEMCAP + L,), jnp.int32),
          pltpu.VMEM((NSUP * SUPCAP + L,), jnp.int32),
          pltpu.VMEM((NSUP * SUPCAP + L,), jnp.int32),
          pltpu.VMEM((NSUP * SUPCAP + L,), jnp.int32),
          pltpu.VMEM((NSUP * SUPCAP + L,), jnp.int32),
          pltpu.VMEM((64, WCOLS), jnp.float32),
          pltpu.VMEM((64, WCOLS), jnp.float32),
          pltpu.VMEM((EXTCAP + L,), jnp.int32),
          pltpu.VMEM((EXTCAP + L,), jnp.int32),
          pltpu.VMEM((ROWSLOTS * 64,), jnp.float32),
          pltpu.VMEM((b_per_w,), jnp.float32),
          pltpu.VMEM((b_per_w,), jnp.float32),
          pltpu.VMEM((L,), jnp.float32),
          pltpu.SMEM((2 * NSUP,), jnp.int32),
          pltpu.SemaphoreType.DMA,
          pltpu.SemaphoreType.DMA,
          pltpu.SemaphoreType.DMA,
          pltpu.SemaphoreType.DMA,
      ],
  )
  ugath, igath, bsum = extract(u_ids, i_ids, uembT, iembT, bias16,
                               user_bias, item_bias)

  half = batch // 2
  u2 = ugath.reshape(half, 2 * d_model)
  i2 = igath.reshape(half, 2 * d_model)
  b2 = bsum.reshape(2, half)
  blk = 2048
  dot = pl.pallas_call(
      functools.partial(_tc_dot_body, d_model),
      grid=(half // blk,),
      in_specs=[
          pl.BlockSpec((blk, 2 * d_model), lambda j: (j, 0)),
          pl.BlockSpec((blk, 2 * d_model), lambda j: (j, 0)),
          pl.BlockSpec((2, blk), lambda j: (0, j)),
      ],
      out_specs=pl.BlockSpec((2, blk), lambda j: (0, j)),
      out_shape=jax.ShapeDtypeStruct((2, half), jnp.float32),
  )
  return dot(u2, i2, b2).reshape(batch)


# speedup vs baseline: 1.2662x; 1.0012x over previous
import functools

import jax
import jax.numpy as jnp
from jax import lax
from jax.experimental import pallas as pl
from jax.experimental.pallas import tpu as pltpu
from jax.experimental.pallas import tpu_sc as plsc

NC = 2
NS = 16
L = 16
NW = NC * NS

TCOLS = 7813
SHARD = 245
WINT = 5
WCOLS = WINT * 128
NWIN = 50
MAXT = TCOLS - WINT
NSUP = 7
SUPT = 35
MEMCAP = 768
SUPCAP = 160
EXTCAP = 96
ROWSLOTS = 32
BATCH = 16384
HALF = BATCH // 2
NCHUNK = BATCH // L


def _extract_body(d_model, b_per_w,
                  u_ids_hbm, i_ids_hbm, uembT_hbm, iembT_hbm, bias_hbm,
                  user_bias_hbm, item_bias_hbm,
                  ugath_hbm, igath_hbm, bsum_hbm,
                  idsu_v, idsi_v, mu_id, mu_pos, mi_id, mi_pos,
                  supu_id, supu_pos, supi_id, supi_pos,
                  win0, win1, ext_col, ext_pos, rowstage,
                  ub_v, ib_v, bias_v,
                  scnt_smem, wsem0, wsem1, rsem, sem_bias):
  wid = lax.axis_index("s") * NC + lax.axis_index("c")
  lanes = lax.iota(jnp.int32, L)
  wins = (win0, win1)
  wsems = (wsem0, wsem1)

  shard_t0 = wid * SHARD
  lo_s = shard_t0 * 128
  hi_s = jnp.minimum((shard_t0 + SHARD) * 128, 1000000)

  def tstart(w):
    return jnp.minimum(shard_t0 + w * WINT, MAXT)

  def fire(table_hbm, w, k):
    off = pl.multiple_of(tstart(w) * 128, 128)
    pltpu.async_copy(table_hbm.at[:, pl.ds(off, WCOLS)], wins[k], wsems[k])

  def drain_win(k):
    pltpu.make_async_copy(
        uembT_hbm.at[:, pl.ds(0, WCOLS)], wins[k], wsems[k]).wait()

  fire(uembT_hbm, 0, 0)
  fire(uembT_hbm, 1, 1)

  pltpu.sync_copy(u_ids_hbm, idsu_v.at[pl.ds(0, BATCH)])
  pltpu.sync_copy(i_ids_hbm, idsi_v.at[pl.ds(0, BATCH)])

  bbase = wid * b_per_w
  cp_ub = pltpu.async_copy(
      user_bias_hbm.at[idsu_v.at[pl.ds(bbase, b_per_w)]], ub_v, sem_bias)
  cp_ib = pltpu.async_copy(
      item_bias_hbm.at[idsi_v.at[pl.ds(bbase, b_per_w)]], ib_v, sem_bias)
  pltpu.sync_copy(bias_hbm, bias_v)

  SEG = MEMCAP // 2
  HCH = NCHUNK // 2

  def scan_chunk(ch, cnts):
    cua, cub, cia, cib = cnts
    outs = []
    for (ids_ref, m_idr, m_posr, c, seg) in (
        (idsu_v, mu_id, mu_pos, cua, 0),
        (idsu_v, mu_id, mu_pos, cub, 1),
        (idsi_v, mi_id, mi_pos, cia, 0),
        (idsi_v, mi_id, mi_pos, cib, 1),
    ):
      chs = ch + seg * HCH
      pos = chs * L + lanes
      v = ids_ref[pl.ds(chs * L, L)]
      m = (v >= lo_s) & (v < hi_s)
      plsc.store_compressed(m_idr.at[pl.ds(seg * SEG + c, L)], v, mask=m)
      plsc.store_compressed(m_posr.at[pl.ds(seg * SEG + c, L)], pos, mask=m)
      outs.append(c + plsc.all_reduce_population_count(m)[0])
    return tuple(outs)

  z = jnp.int32(0)
  cua, cub, cia, cib = lax.fori_loop(0, HCH, scan_chunk, (z, z, z, z),
                                     unroll=2)

  nmax = (jnp.maximum(jnp.maximum(cua, cub), jnp.maximum(cia, cib))
          + L - 1) // L
  for b in range(NSUP):
    blo = lo_s + b * SUPT * 128
    bhi = lo_s + (b + 1) * SUPT * 128

    def sup_seg(j, carry, blo=blo, bhi=bhi, seg=0):
      scu, sci, cu_n, ci_n = carry
      for (m_idr, m_posr, s_id, s_pos, sc, cn, is_u) in (
          (mu_id, mu_pos, supu_id, supu_pos, scu, cu_n, True),
          (mi_id, mi_pos, supi_id, supi_pos, sci, ci_n, False),
      ):
        ids_c = m_idr[pl.ds(seg * SEG + j * L, L)]
        pos_c = m_posr[pl.ds(seg * SEG + j * L, L)]
        m = (ids_c >= blo) & (ids_c < bhi) & (j * L + lanes < cn)
        plsc.store_compressed(s_id.at[pl.ds(b * SUPCAP + sc, L)], ids_c,
                              mask=m)
        plsc.store_compressed(s_pos.at[pl.ds(b * SUPCAP + sc, L)], pos_c,
                              mask=m)
        npc = plsc.all_reduce_population_count(m)[0]
        if is_u:
          scu = sc + npc
        else:
          sci = sc + npc
      return (scu, sci, cu_n, ci_n)

    sa = lax.fori_loop(0, nmax, functools.partial(sup_seg, seg=0),
                       (z, z, cua, cia))
    sb = lax.fori_loop(0, nmax, functools.partial(sup_seg, seg=1),
                       (sa[0], sa[1], cub, cib))
    scnt_smem[b] = sb[0]
    scnt_smem[NSUP + b] = sb[1]

  def run_table(table_hbm, out_hbm, s_id, s_pos, sbase, etot0):
    def process(w, k, etot_in):
      lo = tstart(w) * 128
      sup = (w * WINT) // SUPT

      n_s = scnt_smem[sbase + sup]

      def rescan(j, ec):
        ids_c = s_id[pl.ds(sup * SUPCAP + j * L, L)]
        pos_c = s_pos[pl.ds(sup * SUPCAP + j * L, L)]
        m = (ids_c >= lo) & (ids_c < lo + WCOLS) & (j * L + lanes < n_s)
        plsc.store_compressed(ext_col.at[pl.ds(ec, L)], ids_c - lo, mask=m)
        plsc.store_compressed(ext_pos.at[pl.ds(ec, L)], pos_c, mask=m)
        return ec + plsc.all_reduce_population_count(m)[0]

      ecnt = lax.fori_loop(0, (n_s + L - 1) // L, rescan, jnp.int32(0))

      def extract(e, etot):
        c0 = ext_col[pl.ds(e, L)][0]
        b0 = ext_pos[pl.ds(e, L)][0]
        slot = (etot % ROWSLOTS) * d_model

        @pl.when(etot >= ROWSLOTS)
        def _():
          pltpu.make_async_copy(
              rowstage.at[pl.ds(0, d_model)],
              out_hbm.at[pl.ds(0, d_model)], rsem).wait()

        for dblk in range(d_model // L):
          g = plsc.load_gather(
              wins[k], [dblk * L + lanes, lanes * 0 + c0])
          rowstage[pl.ds(slot + dblk * L, L)] = g
        foff = (b0 % HALF) * (2 * d_model) + (b0 // HALF) * d_model
        pltpu.async_copy(
            rowstage.at[pl.ds(slot, d_model)],
            out_hbm.at[pl.ds(foff, d_model)], rsem)
        return etot + 1

      return lax.fori_loop(0, ecnt, extract, etot_in)

    def pair(p, etot):
      for k in range(2):
        w = p * 2 + k
        drain_win(k)
        etot = process(w, k, etot)
        fire(table_hbm, w + 2, k)
      return etot

    etot = lax.fori_loop(0, NWIN // 2 - 1, pair, etot0)
    for k in range(2):
      w = NWIN - 2 + k
      drain_win(k)
      etot = process(w, k, etot)
    return etot

  cp_ub.wait()
  cp_ib.wait()
  bias_splat = bias_v[...]
  for c in range(b_per_w // L):
    sl = pl.ds(c * L, L)
    ub_v[sl] = ub_v[sl] + ib_v[sl] + bias_splat
  pltpu.sync_copy(ub_v, bsum_hbm.at[pl.ds(bbase, b_per_w)])

  etot = run_table(uembT_hbm, ugath_hbm, supu_id, supu_pos, 0, jnp.int32(0))
  fire(iembT_hbm, 0, 0)
  fire(iembT_hbm, 1, 1)
  etot = run_table(iembT_hbm, igath_hbm, supi_id, supi_pos, NSUP, etot)

  def drain_row(j, carry):
    pltpu.make_async_copy(
        rowstage.at[pl.ds(0, d_model)],
        ugath_hbm.at[pl.ds(0, d_model)], rsem).wait()
    return carry

  lax.fori_loop(0, jnp.minimum(etot, ROWSLOTS), drain_row, jnp.int32(0))


def _tc_dot_body(d_model, u_ref, i_ref, b_ref, o_ref):
  prod = u_ref[...] * i_ref[...]
  a = jnp.sum(prod[:, :d_model], axis=1)
  b = jnp.sum(prod[:, d_model:], axis=1)
  o_ref[...] = b_ref[...] + jnp.stack([a, b], axis=0)


def kernel(u_ids, i_ids, user_emb, item_emb, user_bias, item_bias, bias):
  batch = u_ids.shape[0]
  d_model = user_emb.shape[1]
  b_per_w = batch // NW
  bias16 = jnp.broadcast_to(bias, (L,))
  uembT = user_emb.T
  iembT = item_emb.T

  mesh = plsc.VectorSubcoreMesh(core_axis_name="c", subcore_axis_name="s",
                                num_cores=NC, num_subcores=NS)

  extract = pl.kernel(
      functools.partial(_extract_body, d_model, b_per_w),
      out_type=(jax.ShapeDtypeStruct((batch * d_model,), jnp.float32),
                jax.ShapeDtypeStruct((batch * d_model,), jnp.float32),
                jax.ShapeDtypeStruct((batch,), jnp.float32)),
      mesh=mesh,
      compiler_params=pltpu.CompilerParams(needs_layout_passes=False),
      scratch_types=[
          pltpu.VMEM((BATCH,), jnp.int32),
          pltpu.VMEM((BATCH,), jnp.int32),
          pltpu.VMEM((MEMCAP + L,), jnp.int32),
          pltpu.VMEM((MEMCAP + L,), jnp.int32),
          pltpu.VMEM((MEMCAP + L,), jnp.int32),
          pltpu.VMEM((MEMCAP + L,), jnp.int32),
          pltpu.VMEM((NSUP * SUPCAP + L,), jnp.int32),
          pltpu.VMEM((NSUP * SUPCAP + L,), jnp.int32),
          pltpu.VMEM((NSUP * SUPCAP + L,), jnp.int32),
          pltpu.VMEM((NSUP * SUPCAP + L,), jnp.int32),
          pltpu.VMEM((64, WCOLS), jnp.float32),
          pltpu.VMEM((64, WCOLS), jnp.float32),
          pltpu.VMEM((EXTCAP + L,), jnp.int32),
          pltpu.VMEM((EXTCAP + L,), jnp.int32),
          pltpu.VMEM((ROWSLOTS * 64,), jnp.float32),
          pltpu.VMEM((b_per_w,), jnp.float32),
          pltpu.VMEM((b_per_w,), jnp.float32),
          pltpu.VMEM((L,), jnp.float32),
          pltpu.SMEM((2 * NSUP,), jnp.int32),
          pltpu.SemaphoreType.DMA,
          pltpu.SemaphoreType.DMA,
          pltpu.SemaphoreType.DMA,
          pltpu.SemaphoreType.DMA,
      ],
  )
  ugath, igath, bsum = extract(u_ids, i_ids, uembT, iembT, bias16,
                               user_bias, item_bias)

  half = batch // 2
  u2 = ugath.reshape(half, 2 * d_model)
  i2 = igath.reshape(half, 2 * d_model)
  b2 = bsum.reshape(2, half)
  blk = 2048
  dot = pl.pallas_call(
      functools.partial(_tc_dot_body, d_model),
      grid=(half // blk,),
      in_specs=[
          pl.BlockSpec((blk, 2 * d_model), lambda j: (j, 0)),
          pl.BlockSpec((blk, 2 * d_model), lambda j: (j, 0)),
          pl.BlockSpec((2, blk), lambda j: (0, j)),
      ],
      out_specs=pl.BlockSpec((2, blk), lambda j: (0, j)),
      out_shape=jax.ShapeDtypeStruct((2, half), jnp.float32),
  )
  return dot(u2, i2, b2).reshape(batch)


# speedup vs baseline: 1.2666x; 1.0002x over previous
import functools

import jax
import jax.numpy as jnp
from jax import lax
from jax.experimental import pallas as pl
from jax.experimental.pallas import tpu as pltpu
from jax.experimental.pallas import tpu_sc as plsc

NC = 2
NS = 16
L = 16
NW = NC * NS

TCOLS = 7813
SHARD = 245
WINT = 5
WCOLS = WINT * 128
NWIN = 50
MAXT = TCOLS - WINT
NSUP = 7
SUPT = 35
MEMCAP = 768
SUPCAP = 160
EXTCAP = 96
ROWSLOTS = 32
BATCH = 16384
HALF = BATCH // 2
NCHUNK = BATCH // L


def _extract_body(d_model, b_per_w,
                  u_ids_hbm, i_ids_hbm, uembT_hbm, iembT_hbm, bias_hbm,
                  user_bias_hbm, item_bias_hbm,
                  ugath_hbm, igath_hbm, bsum_hbm,
                  idsu_v, idsi_v, mu_id, mu_pos, mi_id, mi_pos,
                  supu_id, supu_pos, supi_id, supi_pos,
                  win0, win1, ext_col, ext_pos, rowstage,
                  ub_v, ib_v, bias_v,
                  scnt_smem, wsem0, wsem1, rsem, sem_bias):
  wid = lax.axis_index("s") * NC + lax.axis_index("c")
  lanes = lax.iota(jnp.int32, L)
  wins = (win0, win1)
  wsems = (wsem0, wsem1)

  shard_t0 = wid * SHARD
  lo_s = shard_t0 * 128
  hi_s = jnp.minimum((shard_t0 + SHARD) * 128, 1000000)

  def tstart(w):
    return jnp.minimum(shard_t0 + w * WINT, MAXT)

  def fire(table_hbm, w, k):
    off = pl.multiple_of(tstart(w) * 128, 128)
    pltpu.async_copy(table_hbm.at[:, pl.ds(off, WCOLS)], wins[k], wsems[k])

  def drain_win(k):
    pltpu.make_async_copy(
        uembT_hbm.at[:, pl.ds(0, WCOLS)], wins[k], wsems[k]).wait()

  fire(uembT_hbm, 0, 0)
  fire(uembT_hbm, 1, 1)

  pltpu.sync_copy(u_ids_hbm, idsu_v.at[pl.ds(0, BATCH)])
  pltpu.sync_copy(i_ids_hbm, idsi_v.at[pl.ds(0, BATCH)])

  bbase = wid * b_per_w
  cp_ub = pltpu.async_copy(
      user_bias_hbm.at[idsu_v.at[pl.ds(bbase, b_per_w)]], ub_v, sem_bias)
  cp_ib = pltpu.async_copy(
      item_bias_hbm.at[idsi_v.at[pl.ds(bbase, b_per_w)]], ib_v, sem_bias)
  pltpu.sync_copy(bias_hbm, bias_v)

  def scan_chunk(ch, cnts):
    cu, ci = cnts
    pos = ch * L + lanes
    u_c = idsu_v[pl.ds(ch * L, L)]
    mu = (u_c >= lo_s) & (u_c < hi_s)
    plsc.store_compressed(mu_id.at[pl.ds(cu, L)], u_c, mask=mu)
    plsc.store_compressed(mu_pos.at[pl.ds(cu, L)], pos, mask=mu)
    i_c = idsi_v[pl.ds(ch * L, L)]
    mi = (i_c >= lo_s) & (i_c < hi_s)
    plsc.store_compressed(mi_id.at[pl.ds(ci, L)], i_c, mask=mi)
    plsc.store_compressed(mi_pos.at[pl.ds(ci, L)], pos, mask=mi)
    return (cu + plsc.all_reduce_population_count(mu)[0],
            ci + plsc.all_reduce_population_count(mi)[0])

  cntu, cnti = lax.fori_loop(0, NCHUNK, scan_chunk,
                             (jnp.int32(0), jnp.int32(0)), unroll=4)

  for t, (m_id, m_pos, s_id, s_pos, cnt) in enumerate(
      ((mu_id, mu_pos, supu_id, supu_pos, cntu),
       (mi_id, mi_pos, supi_id, supi_pos, cnti))):
    nmemchunk = (cnt + L - 1) // L
    for b in range(NSUP):
      blo = lo_s + b * SUPT * 128
      bhi = lo_s + (b + 1) * SUPT * 128

      def sup_chunk(j, sc, blo=blo, bhi=bhi, b=b,
                    m_id=m_id, m_pos=m_pos, s_id=s_id, s_pos=s_pos, cnt=cnt):
        ids_c = m_id[pl.ds(j * L, L)]
        pos_c = m_pos[pl.ds(j * L, L)]
        m = (ids_c >= blo) & (ids_c < bhi) & (j * L + lanes < cnt)
        plsc.store_compressed(s_id.at[pl.ds(b * SUPCAP + sc, L)], ids_c,
                              mask=m)
        plsc.store_compressed(s_pos.at[pl.ds(b * SUPCAP + sc, L)], pos_c,
                              mask=m)
        return sc + plsc.all_reduce_population_count(m)[0]

      scnt_smem[t * NSUP + b] = lax.fori_loop(0, nmemchunk, sup_chunk,
                                              jnp.int32(0))

  def run_table(table_hbm, out_hbm, s_id, s_pos, sbase, etot0):
    def process(w, k, etot_in):
      lo = tstart(w) * 128
      sup = (w * WINT) // SUPT

      n_s = scnt_smem[sbase + sup]

      def rescan(j, ec):
        ids_c = s_id[pl.ds(sup * SUPCAP + j * L, L)]
        pos_c = s_pos[pl.ds(sup * SUPCAP + j * L, L)]
        m = (ids_c >= lo) & (ids_c < lo + WCOLS) & (j * L + lanes < n_s)
        plsc.store_compressed(ext_col.at[pl.ds(ec, L)], ids_c - lo, mask=m)
        plsc.store_compressed(ext_pos.at[pl.ds(ec, L)], pos_c, mask=m)
        return ec + plsc.all_reduce_population_count(m)[0]

      ecnt = lax.fori_loop(0, (n_s + L - 1) // L, rescan, jnp.int32(0))

      def extract(e, etot):
        c0 = ext_col[pl.ds(e, L)][0]
        b0 = ext_pos[pl.ds(e, L)][0]
        slot = (etot % ROWSLOTS) * d_model

        @pl.when(etot >= ROWSLOTS)
        def _():
          pltpu.make_async_copy(
              rowstage.at[pl.ds(0, d_model)],
              out_hbm.at[pl.ds(0, d_model)], rsem).wait()

        for dblk in range(d_model // L):
          g = plsc.load_gather(
              wins[k], [dblk * L + lanes, lanes * 0 + c0])
          rowstage[pl.ds(slot + dblk * L, L)] = g
        foff = (b0 % HALF) * (2 * d_model) + (b0 // HALF) * d_model
        pltpu.async_copy(
            rowstage.at[pl.ds(slot, d_model)],
            out_hbm.at[pl.ds(foff, d_model)], rsem)
        return etot + 1

      return lax.fori_loop(0, ecnt, extract, etot_in)

    def pair(p, etot):
      for k in range(2):
        w = p * 2 + k
        drain_win(k)
        etot = process(w, k, etot)
        fire(table_hbm, w + 2, k)
      return etot

    etot = lax.fori_loop(0, NWIN // 2 - 1, pair, etot0)
    for k in range(2):
      w = NWIN - 2 + k
      drain_win(k)
      etot = process(w, k, etot)
    return etot

  cp_ub.wait()
  cp_ib.wait()
  bias_splat = bias_v[...]
  for c in range(b_per_w // L):
    sl = pl.ds(c * L, L)
    ub_v[sl] = ub_v[sl] + ib_v[sl] + bias_splat
  pltpu.sync_copy(ub_v, bsum_hbm.at[pl.ds(bbase, b_per_w)])

  etot = run_table(uembT_hbm, ugath_hbm, supu_id, supu_pos, 0, jnp.int32(0))
  fire(iembT_hbm, 0, 0)
  fire(iembT_hbm, 1, 1)
  etot = run_table(iembT_hbm, igath_hbm, supi_id, supi_pos, NSUP, etot)

  def drain_row(j, carry):
    pltpu.make_async_copy(
        rowstage.at[pl.ds(0, d_model)],
        ugath_hbm.at[pl.ds(0, d_model)], rsem).wait()
    return carry

  lax.fori_loop(0, jnp.minimum(etot, ROWSLOTS), drain_row, jnp.int32(0))


def _tc_dot_body(d_model, u_ref, i_ref, b_ref, o_ref):
  prod = u_ref[...] * i_ref[...]
  a = jnp.sum(prod[:, :d_model], axis=1)
  b = jnp.sum(prod[:, d_model:], axis=1)
  o_ref[...] = b_ref[...] + jnp.stack([a, b], axis=0)


def kernel(u_ids, i_ids, user_emb, item_emb, user_bias, item_bias, bias):
  batch = u_ids.shape[0]
  d_model = user_emb.shape[1]
  b_per_w = batch // NW
  bias16 = jnp.broadcast_to(bias, (L,))
  uembT = user_emb.T
  iembT = item_emb.T

  mesh = plsc.VectorSubcoreMesh(core_axis_name="c", subcore_axis_name="s",
                                num_cores=NC, num_subcores=NS)

  extract = pl.kernel(
      functools.partial(_extract_body, d_model, b_per_w),
      out_type=(jax.ShapeDtypeStruct((batch * d_model,), jnp.float32),
                jax.ShapeDtypeStruct((batch * d_model,), jnp.float32),
                jax.ShapeDtypeStruct((batch,), jnp.float32)),
      mesh=mesh,
      compiler_params=pltpu.CompilerParams(needs_layout_passes=False),
      scratch_types=[
          pltpu.VMEM((BATCH,), jnp.int32),
          pltpu.VMEM((BATCH,), jnp.int32),
          pltpu.VMEM((MEMCAP + L,), jnp.int32),
          pltpu.VMEM((MEMCAP + L,), jnp.int32),
          pltpu.VMEM((MEMCAP + L,), jnp.int32),
          pltpu.VMEM((MEMCAP + L,), jnp.int32),
          pltpu.VMEM((NSUP * SUPCAP + L,), jnp.int32),
          pltpu.VMEM((NSUP * SUPCAP + L,), jnp.int32),
          pltpu.VMEM((NSUP * SUPCAP + L,), jnp.int32),
          pltpu.VMEM((NSUP * SUPCAP + L,), jnp.int32),
          pltpu.VMEM((64, WCOLS), jnp.float32),
          pltpu.VMEM((64, WCOLS), jnp.float32),
          pltpu.VMEM((EXTCAP + L,), jnp.int32),
          pltpu.VMEM((EXTCAP + L,), jnp.int32),
          pltpu.VMEM((ROWSLOTS * 64,), jnp.float32),
          pltpu.VMEM((b_per_w,), jnp.float32),
          pltpu.VMEM((b_per_w,), jnp.float32),
          pltpu.VMEM((L,), jnp.float32),
          pltpu.SMEM((2 * NSUP,), jnp.int32),
          pltpu.SemaphoreType.DMA,
          pltpu.SemaphoreType.DMA,
          pltpu.SemaphoreType.DMA,
          pltpu.SemaphoreType.DMA,
      ],
  )
  ugath, igath, bsum = extract(u_ids, i_ids, uembT, iembT, bias16,
                               user_bias, item_bias)

  half = batch // 2
  u2 = ugath.reshape(half, 2 * d_model)
  i2 = igath.reshape(half, 2 * d_model)
  b2 = bsum.reshape(2, half)
  blk = 2048
  dot = pl.pallas_call(
      functools.partial(_tc_dot_body, d_model),
      grid=(half // blk,),
      in_specs=[
          pl.BlockSpec((blk, 2 * d_model), lambda j: (j, 0)),
          pl.BlockSpec((blk, 2 * d_model), lambda j: (j, 0)),
          pl.BlockSpec((2, blk), lambda j: (0, j)),
      ],
      out_specs=pl.BlockSpec((2, blk), lambda j: (0, j)),
      out_shape=jax.ShapeDtypeStruct((2, half), jnp.float32),
  )
  return dot(u2, i2, b2).reshape(batch)
